# Initial kernel scaffold; baseline (speedup 1.0000x reference)
import jax
import jax.numpy as jnp
from jax.experimental import pallas as pl


def kernel(x, edge_index, W1, a1_src, a1_dst, b1, W2, a2_src, a2_dst, b2, W3, a3_src, a3_dst, b3):
    raise NotImplementedError("write your pallas kernel here")


# speedup vs baseline: 6.3800x; 6.3800x over previous
import functools

import jax
import jax.numpy as jnp
from jax import lax
from jax.experimental import pallas as pl
from jax.experimental.pallas import tpu as pltpu
from jax.experimental.pallas import tpu_sc as plsc

N = 10000
NPAD = 10240
D = 512
NCHUNK = 4
DC = 128
CPSC = NCHUNK // 2
EPAD = 172032
ET = EPAD // 16
B = 64
NB = ET // B
RPT = NPAD // 16
NHALF = NPAD // 2
NP = RPT // B


def _mm_body(chunked_in, x_ref, w_ref, a2_ref, b_ref, h4_ref, aa_ref):
    if chunked_in:
        x = jnp.concatenate([x_ref[c] for c in range(NCHUNK)], axis=1)
        x = jnp.maximum(x + b_ref[0], 0.0)
    else:
        x = x_ref[...]
    h = jnp.dot(x, w_ref[...],
                preferred_element_type=jnp.float32,
                precision=lax.Precision.HIGHEST)
    for c in range(NCHUNK):
        h4_ref[c] = h[:, c * DC:(c + 1) * DC]
    aa_ref[...] = jnp.dot(h, a2_ref[...],
                          preferred_element_type=jnp.float32,
                          precision=lax.Precision.HIGHEST)


def _matmul(x_in, W, a_src, a_dst, b_prev):
    chunked_in = x_in.ndim == 3
    a2 = jnp.stack([a_src, a_dst] + [jnp.zeros_like(a_src)] * 6, axis=1)
    blk = 640
    grid = NPAD // blk
    if chunked_in:
        x_spec = pl.BlockSpec((NCHUNK, blk, DC), lambda i: (0, i, 0))
        din = D
    else:
        din = x_in.shape[1]
        x_spec = pl.BlockSpec((blk, din), lambda i: (i, 0))
    h4, aa = pl.pallas_call(
        functools.partial(_mm_body, chunked_in),
        grid=(grid,),
        in_specs=[
            x_spec,
            pl.BlockSpec((din, D), lambda i: (0, 0)),
            pl.BlockSpec((D, 8), lambda i: (0, 0)),
            pl.BlockSpec((1, D), lambda i: (0, 0)),
        ],
        out_specs=[
            pl.BlockSpec((NCHUNK, blk, DC), lambda i: (0, i, 0)),
            pl.BlockSpec((blk, 8), lambda i: (i, 0)),
        ],
        out_shape=[
            jax.ShapeDtypeStruct((NCHUNK, NPAD, DC), jnp.float32),
            jax.ShapeDtypeStruct((NPAD, 8), jnp.float32),
        ],
    )(x_in, W, a2, b_prev.reshape(1, D))
    return jnp.reshape(h4, (NCHUNK * NPAD, DC)), aa[:, 0], aa[:, 1]


def _jk_body(u1, u2, u3, b1, b2, b3, o):
    outs = []
    for u, b in ((u1, b1), (u2, b2), (u3, b3)):
        xc = jnp.concatenate([u[c] for c in range(NCHUNK)], axis=1)
        outs.append(jnp.maximum(xc + b[0], 0.0))
    o[...] = jnp.maximum(jnp.maximum(outs[0], outs[1]), outs[2])


def _jk_max(u1, u2, u3, b1, b2, b3):
    blk = 640
    grid = NPAD // blk
    uspec = pl.BlockSpec((NCHUNK, blk, DC), lambda i: (0, i, 0))
    bspec = pl.BlockSpec((1, D), lambda i: (0, 0))
    return pl.pallas_call(
        _jk_body,
        grid=(grid,),
        in_specs=[uspec, uspec, uspec, bspec, bspec, bspec],
        out_specs=pl.BlockSpec((blk, D), lambda i: (i, 0)),
        out_shape=jax.ShapeDtypeStruct((NPAD, D), jnp.float32),
    )(u1, u2, u3, b1.reshape(1, D), b2.reshape(1, D), b3.reshape(1, D))


def _sc_body(srcH, dstH, asH, adH, h4H, outH,
             sidx1d, dst1d, w1d, abuf, rowbuf, dslice, gbuf, didxb,
             acc, sem):
    c = lax.axis_index("c")
    s = lax.axis_index("s")
    zv = jnp.zeros((16,), jnp.float32)
    ebase = s * ET

    pltpu.sync_copy(srcH.at[s], sidx1d)
    pltpu.sync_copy(dstH.at[s], dst1d)

    for (arr, idx) in ((asH, sidx1d), (adH, dst1d)):
        for half in range(2):
            pltpu.sync_copy(arr.at[pl.ds(half * NHALF, NHALF)], abuf)
            lo = half * NHALF

            def gsum(i, _, idx=idx, lo=lo, first=(arr is asH and half == 0)):
                sl = pl.ds(i * 16, 16)
                iv = idx[sl] - lo
                ivc = jnp.minimum(jnp.maximum(iv, 0), NHALF - 1)
                av = plsc.load_gather(abuf, [ivc])
                av = jnp.where((iv >= 0) & (iv < NHALF), av, 0.0)
                if first:
                    w1d[sl] = av
                else:
                    w1d[sl] = w1d[sl] + av
                return 0
            lax.fori_loop(0, ET // 16, gsum, 0)

    def wfin(i, _):
        sl = pl.ds(i * 16, 16)
        t = w1d[sl]
        t = jnp.maximum(t, 0.2 * t)
        w = jnp.exp(t)
        gid = ebase + i * 16 + lax.iota(jnp.int32, 16)
        w1d[sl] = jnp.where(gid < N + 160000, w, 0.0)
        return 0
    lax.fori_loop(0, ET // 16, wfin, 0)

    def zrow(r, _):
        for j in range(DC // 16):
            rowbuf[r, pl.ds(j * 16, 16)] = zv
        return 0

    def zacc(p, _):
        pltpu.sync_copy(rowbuf, acc.at[pl.ds(s * RPT + p * B, B)])
        return 0

    lax.fori_loop(0, B, zrow, 0)
    lax.fori_loop(0, NP, zacc, 0)
    plsc.subcore_barrier()

    lane0 = lax.iota(jnp.int32, 16) == 0

    def fill_didx(b):
        for j in range(B // 16):
            sl = pl.ds(j * 16, 16)
            didxb[sl] = dst1d[pl.ds(b * B + j * 16, 16)]

    def dbatch(b, _):
        fill_didx(b)

        def drow(r, _):
            wsp = plsc.load_gather(w1d, [jnp.full((16,), b * B, jnp.int32)
                                         + r])
            rowbuf[r, pl.ds(0, 16)] = jnp.where(lane0, wsp, 0.0)
            return 0
        lax.fori_loop(0, B, drow, 0)
        pltpu.sync_copy(rowbuf, acc.at[didxb], add=True)
        return 0
    lax.fori_loop(0, NB, dbatch, 0)
    plsc.subcore_barrier()

    def dext(p, _):
        pltpu.sync_copy(acc.at[pl.ds(s * RPT + p * B, B)], rowbuf)
        for k in range(B // 16):
            ridx = k * 16 + lax.iota(jnp.int32, 16)
            dv = plsc.load_gather(rowbuf, [ridx, jnp.zeros((16,), jnp.int32)])
            dslice[pl.ds(p * B + k * 16, 16)] = dv
        return 0
    lax.fori_loop(0, NP, dext, 0)

    def do_chunk(q, _):
        chunk = CPSC * c + q

        lax.fori_loop(0, B, zrow, 0)
        lax.fori_loop(0, NP, zacc, 0)
        plsc.subcore_barrier()

        def batch(b, _):
            fill_didx(b)
            for j in range(B // 16):
                sl = pl.ds(j * 16, 16)
                gbuf[sl] = sidx1d[pl.ds(b * B + j * 16, 16)] + chunk * NPAD
            pltpu.async_copy(h4H.at[gbuf], rowbuf, sem).wait()

            def scale(r, _):
                wsp = plsc.load_gather(w1d, [jnp.full((16,), b * B, jnp.int32)
                                             + r])
                for j in range(DC // 16):
                    sl = pl.ds(j * 16, 16)
                    rowbuf[r, sl] = rowbuf[r, sl] * wsp
                return 0
            lax.fori_loop(0, B, scale, 0)
            pltpu.sync_copy(rowbuf, acc.at[didxb], add=True)
            return 0
        lax.fori_loop(0, NB, batch, 0)
        plsc.subcore_barrier()

        def norm(p, _):
            row0 = s * RPT + p * B
            pltpu.sync_copy(acc.at[pl.ds(row0, B)], rowbuf)

            def nrow(r, _):
                dsp = plsc.load_gather(dslice, [jnp.full((16,), p * B,
                                                         jnp.int32) + r])
                inv = 1.0 / (dsp + 1e-16)
                for j in range(DC // 16):
                    sl = pl.ds(j * 16, 16)
                    rowbuf[r, sl] = rowbuf[r, sl] * inv
                return 0
            lax.fori_loop(0, B, nrow, 0)
            pltpu.sync_copy(rowbuf, outH.at[pl.ds(chunk * NPAD + row0, B)])
            return 0
        lax.fori_loop(0, NP, norm, 0)
        plsc.subcore_barrier()
        return 0
    lax.fori_loop(0, CPSC, do_chunk, 0)


def _sc_layer(srcH, dstH, a_s, a_d, h4):
    mesh = plsc.VectorSubcoreMesh(core_axis_name="c", subcore_axis_name="s")
    kfn = pl.kernel(
        _sc_body,
        out_type=jax.ShapeDtypeStruct((NCHUNK * NPAD, DC), jnp.float32),
        mesh=mesh,
        scratch_types=[
            pltpu.VMEM((ET,), jnp.int32),
            pltpu.VMEM((ET,), jnp.int32),
            pltpu.VMEM((ET,), jnp.float32),
            pltpu.VMEM((NHALF,), jnp.float32),
            pltpu.VMEM((B, DC), jnp.float32),
            pltpu.VMEM((RPT,), jnp.float32),
            pltpu.VMEM((B,), jnp.int32),
            pltpu.VMEM((B,), jnp.int32),
            pltpu.VMEM_SHARED((NPAD, DC), jnp.float32),
            pltpu.SemaphoreType.DMA,
        ],
        compiler_params=pltpu.CompilerParams(needs_layout_passes=False),
    )
    return kfn(srcH, dstH, a_s, a_d, h4)


def kernel(x, edge_index, W1, a1_src, a1_dst, b1,
           W2, a2_src, a2_dst, b2, W3, a3_src, a3_dst, b3):
    ei = edge_index.astype(jnp.int32)
    loops = jnp.arange(N, dtype=jnp.int32)
    src = jnp.concatenate([ei[0], loops])
    dst = jnp.concatenate([ei[1], loops])
    src = jnp.pad(src, (0, EPAD - src.shape[0]))
    dst = jnp.pad(dst, (0, EPAD - dst.shape[0]))
    srcH = src.reshape(16, ET)
    dstH = dst.reshape(16, ET)

    x_pad = jnp.pad(x, ((0, NPAD - N), (0, 0)))

    h4, a_s, a_d = _matmul(x_pad, W1, a1_src, a1_dst, b1)
    u1 = _sc_layer(srcH, dstH, a_s, a_d, h4)
    u1c = u1.reshape(NCHUNK, NPAD, DC)
    h4, a_s, a_d = _matmul(u1c, W2, a2_src, a2_dst, b1)
    u2 = _sc_layer(srcH, dstH, a_s, a_d, h4)
    u2c = u2.reshape(NCHUNK, NPAD, DC)
    h4, a_s, a_d = _matmul(u2c, W3, a3_src, a3_dst, b2)
    u3 = _sc_layer(srcH, dstH, a_s, a_d, h4)
    u3c = u3.reshape(NCHUNK, NPAD, DC)

    out = _jk_max(u1c, u2c, u3c, b1, b2, b3)
    return out[:N]


# speedup vs baseline: 8.6361x; 1.3536x over previous
import functools

import jax
import jax.numpy as jnp
from jax import lax
from jax.experimental import pallas as pl
from jax.experimental.pallas import tpu as pltpu
from jax.experimental.pallas import tpu_sc as plsc

N = 10000
NPAD = 10240
D = 512
NCHUNK = 4
DC = 128
CPSC = NCHUNK // 2
EPAD = 172032
ET = EPAD // 16
B = 64
NB = ET // B
NT = NB // 3
RPT = NPAD // 16
NP = RPT // B
NE = 1280
PACK = 16384


def _mm_body(chunked_in, x_ref, w_ref, a2_ref, b_ref, h4_ref, aa_ref):
    if chunked_in:
        x = jnp.concatenate([x_ref[c] for c in range(NCHUNK)], axis=1)
        x = jnp.maximum(x + b_ref[0], 0.0)
    else:
        x = x_ref[...]
    h = jnp.dot(x, w_ref[...],
                preferred_element_type=jnp.float32,
                precision=lax.Precision.HIGHEST)
    for c in range(NCHUNK):
        h4_ref[c] = h[:, c * DC:(c + 1) * DC]
    aa_ref[...] = jnp.dot(h, a2_ref[...],
                          preferred_element_type=jnp.float32,
                          precision=lax.Precision.HIGHEST)


def _matmul(x_in, W, a_src, a_dst, b_prev):
    chunked_in = x_in.ndim == 3
    a2 = jnp.stack([a_src, a_dst] + [jnp.zeros_like(a_src)] * 6, axis=1)
    blk = 640
    grid = NPAD // blk
    if chunked_in:
        x_spec = pl.BlockSpec((NCHUNK, blk, DC), lambda i: (0, i, 0))
        din = D
    else:
        din = x_in.shape[1]
        x_spec = pl.BlockSpec((blk, din), lambda i: (i, 0))
    h4, aa = pl.pallas_call(
        functools.partial(_mm_body, chunked_in),
        grid=(grid,),
        in_specs=[
            x_spec,
            pl.BlockSpec((din, D), lambda i: (0, 0)),
            pl.BlockSpec((D, 8), lambda i: (0, 0)),
            pl.BlockSpec((1, D), lambda i: (0, 0)),
        ],
        out_specs=[
            pl.BlockSpec((NCHUNK, blk, DC), lambda i: (0, i, 0)),
            pl.BlockSpec((blk, 8), lambda i: (i, 0)),
        ],
        out_shape=[
            jax.ShapeDtypeStruct((NCHUNK, NPAD, DC), jnp.float32),
            jax.ShapeDtypeStruct((NPAD, 8), jnp.float32),
        ],
    )(x_in, W, a2, b_prev.reshape(1, D))
    return jnp.reshape(h4, (NCHUNK * NPAD, DC)), aa[:, 0], aa[:, 1]


def _jk_body(u1, u2, u3, b1, b2, b3, o):
    outs = []
    for u, b in ((u1, b1), (u2, b2), (u3, b3)):
        xc = jnp.concatenate([u[c] for c in range(NCHUNK)], axis=1)
        outs.append(jnp.maximum(xc + b[0], 0.0))
    o[...] = jnp.maximum(jnp.maximum(outs[0], outs[1]), outs[2])


def _jk_max(u1, u2, u3, b1, b2, b3):
    blk = 640
    grid = NPAD // blk
    uspec = pl.BlockSpec((NCHUNK, blk, DC), lambda i: (0, i, 0))
    bspec = pl.BlockSpec((1, D), lambda i: (0, 0))
    return pl.pallas_call(
        _jk_body,
        grid=(grid,),
        in_specs=[uspec, uspec, uspec, bspec, bspec, bspec],
        out_specs=pl.BlockSpec((blk, D), lambda i: (i, 0)),
        out_shape=jax.ShapeDtypeStruct((NPAD, D), jnp.float32),
    )(u1, u2, u3, b1.reshape(1, D), b2.reshape(1, D), b3.reshape(1, D))


def _sc_body(epkH, asH, adH, h4H, outH,
             epk1d, w1d, abuf, rb0, rb1, rb2, dslice, gbuf, dx0, dx1, dx2,
             acc, gs0, gs1, gs2, ss0, ss1, ss2):
    c = lax.axis_index("c")
    s = lax.axis_index("s")
    zv = jnp.zeros((16,), jnp.float32)
    ebase = s * ET
    rbufs = (rb0, rb1, rb2)
    didxs = (dx0, dx1, dx2)
    gsems = (gs0, gs1, gs2)
    ssems = (ss0, ss1, ss2)

    pltpu.sync_copy(epkH.at[s], epk1d)

    for ai, arr in ((0, asH), (1, adH)):
        for piece in range(NPAD // NE):
            pltpu.sync_copy(arr.at[pl.ds(piece * NE, NE)], abuf)
            lo = piece * NE
            first = ai == 0 and piece == 0

            def gsum(i, _, ai=ai, lo=lo, first=first):
                sl = pl.ds(i * 16, 16)
                ev = epk1d[sl]
                nv = (ev >> 14) if ai == 0 else (ev & (PACK - 1))
                iv = nv - lo
                ivc = jnp.minimum(jnp.maximum(iv, 0), NE - 1)
                av = plsc.load_gather(abuf, [ivc])
                av = jnp.where((iv >= 0) & (iv < NE), av, 0.0)
                if first:
                    w1d[sl] = av
                else:
                    w1d[sl] = w1d[sl] + av
                return 0
            lax.fori_loop(0, ET // 16, gsum, 0)

    def wfin(i, _):
        sl = pl.ds(i * 16, 16)
        t = w1d[sl]
        t = jnp.maximum(t, 0.2 * t)
        w = jnp.exp(t)
        gid = ebase + i * 16 + lax.iota(jnp.int32, 16)
        w1d[sl] = jnp.where(gid < N + 160000, w, 0.0)
        return 0
    lax.fori_loop(0, ET // 16, wfin, 0)

    def zrow(rbuf):
        def f(r, _):
            for j in range(DC // 16):
                rbuf[r, pl.ds(j * 16, 16)] = zv
            return 0
        return f

    def zacc(p, _):
        pltpu.sync_copy(rb0, acc.at[pl.ds(s * RPT + p * B, B)])
        return 0

    def fill_didx(b, par):
        for j in range(B // 16):
            sl = pl.ds(j * 16, 16)
            ev = epk1d[pl.ds(b * B + j * 16, 16)]
            didxs[par][sl] = ev & (PACK - 1)

    def wsplat(b, r):
        return plsc.load_gather(w1d, [jnp.full((16,), b * B, jnp.int32) + r])

    def scat_start(par):
        pltpu.async_copy(rbufs[par], acc.at[didxs[par]], ssems[par], add=True)

    def scat_wait(par):
        pltpu.make_async_copy(rbufs[par], acc.at[didxs[par]],
                              ssems[par]).wait()

    lax.fori_loop(0, B, zrow(rb0), 0)
    lax.fori_loop(0, NP, zacc, 0)
    plsc.subcore_barrier()

    lane0 = lax.iota(jnp.int32, 16) == 0
    for par in range(3):
        lax.fori_loop(0, B, zrow(rbufs[par]), 0)

    def dtriple(t, _):
        for par in range(3):
            b = 3 * t + par

            @pl.when(t > 0)
            def _():
                scat_wait(par)
            fill_didx(b, par)

            def drow(r, _, par=par, b=b):
                rbufs[par][r, pl.ds(0, 16)] = jnp.where(lane0, wsplat(b, r),
                                                        0.0)
                return 0
            lax.fori_loop(0, B, drow, 0)
            scat_start(par)
        return 0
    lax.fori_loop(0, NT, dtriple, 0)
    for par in range(3):
        scat_wait(par)
    plsc.subcore_barrier()

    def dext(p, _):
        pltpu.sync_copy(acc.at[pl.ds(s * RPT + p * B, B)], rb0)
        for k in range(B // 16):
            ridx = k * 16 + lax.iota(jnp.int32, 16)
            dv = plsc.load_gather(rb0, [ridx, jnp.zeros((16,), jnp.int32)])
            dslice[pl.ds(p * B + k * 16, 16)] = dv
        return 0
    lax.fori_loop(0, NP, dext, 0)

    def gath_start(b, par, chunk):
        for j in range(B // 16):
            sl = pl.ds(j * 16, 16)
            ev = epk1d[pl.ds(b * B + j * 16, 16)]
            gbuf[sl] = (ev >> 14) + chunk * NPAD
        fill_didx(b, par)
        pltpu.async_copy(h4H.at[gbuf], rbufs[par], gsems[par])

    def gath_wait(par):
        pltpu.make_async_copy(h4H.at[gbuf], rbufs[par], gsems[par]).wait()

    def do_chunk(q, _):
        chunk = CPSC * c + q

        lax.fori_loop(0, B, zrow(rb0), 0)
        lax.fori_loop(0, NP, zacc, 0)
        plsc.subcore_barrier()

        gath_start(0, 0, chunk)

        def triple(t, _):
            for par in range(3):
                b = 3 * t + par
                nxt = (par + 1) % 3
                gath_wait(par)

                if par == 2:
                    scat_wait(nxt)
                else:
                    @pl.when(t > 0)
                    def _():
                        scat_wait(nxt)

                if par < 2:
                    gath_start(b + 1, nxt, chunk)
                else:
                    @pl.when(t < NT - 1)
                    def _():
                        gath_start(b + 1, nxt, chunk)

                def scale(r, _, par=par, b=b):
                    wsp = wsplat(b, r)
                    for j in range(DC // 16):
                        sl = pl.ds(j * 16, 16)
                        rbufs[par][r, sl] = rbufs[par][r, sl] * wsp
                    return 0
                lax.fori_loop(0, B, scale, 0)
                scat_start(par)
            return 0
        lax.fori_loop(0, NT, triple, 0)
        for par in (1, 2):
            scat_wait(par)
        plsc.subcore_barrier()

        def norm(p, _):
            row0 = s * RPT + p * B
            pltpu.sync_copy(acc.at[pl.ds(row0, B)], rb0)

            def nrow(r, _):
                dsp = plsc.load_gather(dslice, [jnp.full((16,), p * B,
                                                         jnp.int32) + r])
                inv = 1.0 / (dsp + 1e-16)
                for j in range(DC // 16):
                    sl = pl.ds(j * 16, 16)
                    rb0[r, sl] = rb0[r, sl] * inv
                return 0
            lax.fori_loop(0, B, nrow, 0)
            pltpu.sync_copy(rb0, outH.at[pl.ds(chunk * NPAD + row0, B)])
            return 0
        lax.fori_loop(0, NP, norm, 0)
        plsc.subcore_barrier()
        return 0
    lax.fori_loop(0, CPSC, do_chunk, 0)


def _sc_layer(epkH, a_s, a_d, h4):
    mesh = plsc.VectorSubcoreMesh(core_axis_name="c", subcore_axis_name="s")
    kfn = pl.kernel(
        _sc_body,
        out_type=jax.ShapeDtypeStruct((NCHUNK * NPAD, DC), jnp.float32),
        mesh=mesh,
        scratch_types=[
            pltpu.VMEM((ET,), jnp.int32),
            pltpu.VMEM((ET,), jnp.float32),
            pltpu.VMEM((NE,), jnp.float32),
            pltpu.VMEM((B, DC), jnp.float32),
            pltpu.VMEM((B, DC), jnp.float32),
            pltpu.VMEM((B, DC), jnp.float32),
            pltpu.VMEM((RPT,), jnp.float32),
            pltpu.VMEM((B,), jnp.int32),
            pltpu.VMEM((B,), jnp.int32),
            pltpu.VMEM((B,), jnp.int32),
            pltpu.VMEM((B,), jnp.int32),
            pltpu.VMEM_SHARED((NPAD, DC), jnp.float32),
            pltpu.SemaphoreType.DMA,
            pltpu.SemaphoreType.DMA,
            pltpu.SemaphoreType.DMA,
            pltpu.SemaphoreType.DMA,
            pltpu.SemaphoreType.DMA,
            pltpu.SemaphoreType.DMA,
        ],
        compiler_params=pltpu.CompilerParams(needs_layout_passes=False),
    )
    return kfn(epkH, a_s, a_d, h4)


def kernel(x, edge_index, W1, a1_src, a1_dst, b1,
           W2, a2_src, a2_dst, b2, W3, a3_src, a3_dst, b3):
    ei = edge_index.astype(jnp.int32)
    loops = jnp.arange(N, dtype=jnp.int32)
    src = jnp.concatenate([ei[0], loops])
    dst = jnp.concatenate([ei[1], loops])
    src = jnp.pad(src, (0, EPAD - src.shape[0]))
    dst = jnp.pad(dst, (0, EPAD - dst.shape[0]))
    epkH = (src * PACK + dst).reshape(16, ET)

    x_pad = jnp.pad(x, ((0, NPAD - N), (0, 0)))

    h4, a_s, a_d = _matmul(x_pad, W1, a1_src, a1_dst, b1)
    u1 = _sc_layer(epkH, a_s, a_d, h4)
    u1c = u1.reshape(NCHUNK, NPAD, DC)
    h4, a_s, a_d = _matmul(u1c, W2, a2_src, a2_dst, b1)
    u2 = _sc_layer(epkH, a_s, a_d, h4)
    u2c = u2.reshape(NCHUNK, NPAD, DC)
    h4, a_s, a_d = _matmul(u2c, W3, a3_src, a3_dst, b2)
    u3 = _sc_layer(epkH, a_s, a_d, h4)
    u3c = u3.reshape(NCHUNK, NPAD, DC)

    out = _jk_max(u1c, u2c, u3c, b1, b2, b3)
    return out[:N]


# speedup vs baseline: 8.7891x; 1.0177x over previous
import functools

import jax
import jax.numpy as jnp
from jax import lax
from jax.experimental import pallas as pl
from jax.experimental.pallas import tpu as pltpu
from jax.experimental.pallas import tpu_sc as plsc

N = 10000
NPAD = 10240
D = 512
NCHUNK = 4
DC = 128
CPSC = NCHUNK // 2
EPAD = 172032
ET = EPAD // 16
B = 64
NB = ET // B
NT = NB // 3
RPT = NPAD // 16
NP = RPT // B
NE = 1280
PACK = 16384


def _mm_body(chunked_in, x_ref, w_ref, a2_ref, b_ref, h4_ref, aa_ref):
    if chunked_in:
        x = jnp.concatenate([x_ref[c] for c in range(NCHUNK)], axis=1)
        x = jnp.maximum(x + b_ref[0], 0.0)
    else:
        x = x_ref[...]
    h = jnp.dot(x, w_ref[...],
                preferred_element_type=jnp.float32,
                precision=lax.Precision.HIGHEST)
    for c in range(NCHUNK):
        h4_ref[c] = h[:, c * DC:(c + 1) * DC]
    aa_ref[...] = jnp.dot(h, a2_ref[...],
                          preferred_element_type=jnp.float32,
                          precision=lax.Precision.HIGHEST)


def _matmul(x_in, W, a_src, a_dst, b_prev):
    chunked_in = x_in.ndim == 3
    a2 = jnp.stack([a_src, a_dst] + [jnp.zeros_like(a_src)] * 6, axis=1)
    blk = 640
    grid = NPAD // blk
    if chunked_in:
        x_spec = pl.BlockSpec((NCHUNK, blk, DC), lambda i: (0, i, 0))
        din = D
    else:
        din = x_in.shape[1]
        x_spec = pl.BlockSpec((blk, din), lambda i: (i, 0))
    h4, aa = pl.pallas_call(
        functools.partial(_mm_body, chunked_in),
        grid=(grid,),
        in_specs=[
            x_spec,
            pl.BlockSpec((din, D), lambda i: (0, 0)),
            pl.BlockSpec((D, 8), lambda i: (0, 0)),
            pl.BlockSpec((1, D), lambda i: (0, 0)),
        ],
        out_specs=[
            pl.BlockSpec((NCHUNK, blk, DC), lambda i: (0, i, 0)),
            pl.BlockSpec((blk, 8), lambda i: (i, 0)),
        ],
        out_shape=[
            jax.ShapeDtypeStruct((NCHUNK, NPAD, DC), jnp.float32),
            jax.ShapeDtypeStruct((NPAD, 8), jnp.float32),
        ],
    )(x_in, W, a2, b_prev.reshape(1, D))
    return jnp.reshape(h4, (NCHUNK * NPAD, DC)), aa[:, 0], aa[:, 1]


def _jk_body(u1, u2, u3, b1, b2, b3, o):
    outs = []
    for u, b in ((u1, b1), (u2, b2), (u3, b3)):
        xc = jnp.concatenate([u[c] for c in range(NCHUNK)], axis=1)
        outs.append(jnp.maximum(xc + b[0], 0.0))
    o[...] = jnp.maximum(jnp.maximum(outs[0], outs[1]), outs[2])


def _jk_max(u1, u2, u3, b1, b2, b3):
    blk = 640
    grid = NPAD // blk
    uspec = pl.BlockSpec((NCHUNK, blk, DC), lambda i: (0, i, 0))
    bspec = pl.BlockSpec((1, D), lambda i: (0, 0))
    return pl.pallas_call(
        _jk_body,
        grid=(grid,),
        in_specs=[uspec, uspec, uspec, bspec, bspec, bspec],
        out_specs=pl.BlockSpec((blk, D), lambda i: (i, 0)),
        out_shape=jax.ShapeDtypeStruct((NPAD, D), jnp.float32),
    )(u1, u2, u3, b1.reshape(1, D), b2.reshape(1, D), b3.reshape(1, D))


def _sc_body(epkH, asH, adH, h4H, outH,
             epk1d, w1d, abuf, rb0, rb1, rb2, dslice, gbuf, dx0, dx1, dx2,
             acc, gs0, gs1, gs2, ss0, ss1, ss2):
    c = lax.axis_index("c")
    s = lax.axis_index("s")
    zv = jnp.zeros((16,), jnp.float32)
    ebase = s * ET
    rbufs = (rb0, rb1, rb2)
    didxs = (dx0, dx1, dx2)
    gsems = (gs0, gs1, gs2)
    ssems = (ss0, ss1, ss2)

    pltpu.sync_copy(epkH.at[s], epk1d)

    for ai, arr in ((0, asH), (1, adH)):
        for piece in range(NPAD // NE):
            pltpu.sync_copy(arr.at[pl.ds(piece * NE, NE)], abuf)
            lo = piece * NE
            first = ai == 0 and piece == 0

            def gsum(i, _, ai=ai, lo=lo, first=first):
                sl = pl.ds(i * 16, 16)
                ev = epk1d[sl]
                nv = (ev >> 14) if ai == 0 else (ev & (PACK - 1))
                iv = nv - lo
                ivc = jnp.minimum(jnp.maximum(iv, 0), NE - 1)
                av = plsc.load_gather(abuf, [ivc])
                av = jnp.where((iv >= 0) & (iv < NE), av, 0.0)
                if first:
                    w1d[sl] = av
                else:
                    w1d[sl] = w1d[sl] + av
                return 0
            lax.fori_loop(0, ET // 16, gsum, 0)

    def wfin(i, _):
        sl = pl.ds(i * 16, 16)
        t = w1d[sl]
        t = jnp.maximum(t, 0.2 * t)
        w = jnp.exp(t)
        gid = ebase + i * 16 + lax.iota(jnp.int32, 16)
        w1d[sl] = jnp.where(gid < N + 160000, w, 0.0)
        return 0
    lax.fori_loop(0, ET // 16, wfin, 0)

    def zrow(rbuf):
        def f(r, _):
            for j in range(DC // 16):
                rbuf[r, pl.ds(j * 16, 16)] = zv
            return 0
        return f

    def zacc(p, _):
        pltpu.sync_copy(rb0, acc.at[pl.ds(s * RPT + p * B, B)])
        return 0

    def fill_didx(b, par):
        for j in range(B // 16):
            sl = pl.ds(j * 16, 16)
            ev = epk1d[pl.ds(b * B + j * 16, 16)]
            didxs[par][sl] = ev & (PACK - 1)

    def wsplat(b, r):
        return plsc.load_gather(w1d, [jnp.full((16,), b * B, jnp.int32) + r])

    def scat_start(par):
        pltpu.async_copy(rbufs[par], acc.at[didxs[par]], ssems[par], add=True)

    def scat_wait(par):
        pltpu.make_async_copy(rbufs[par], acc.at[didxs[par]],
                              ssems[par]).wait()

    lax.fori_loop(0, B, zrow(rb0), 0)
    lax.fori_loop(0, NP, zacc, 0)
    plsc.subcore_barrier()

    lane0 = lax.iota(jnp.int32, 16) == 0
    for par in range(3):
        lax.fori_loop(0, B, zrow(rbufs[par]), 0)

    def dtriple(t, _):
        for par in range(3):
            b = 3 * t + par

            @pl.when(t > 0)
            def _():
                scat_wait(par)
            fill_didx(b, par)

            @plsc.parallel_loop(0, B, 1, unroll=4)
            def drow(r, par=par, b=b):
                rbufs[par][r, pl.ds(0, 16)] = jnp.where(lane0, wsplat(b, r),
                                                        0.0)
            scat_start(par)
        return 0
    lax.fori_loop(0, NT, dtriple, 0)
    for par in range(3):
        scat_wait(par)
    plsc.subcore_barrier()

    def dext(p, _):
        pltpu.sync_copy(acc.at[pl.ds(s * RPT + p * B, B)], rb0)
        for k in range(B // 16):
            ridx = k * 16 + lax.iota(jnp.int32, 16)
            dv = plsc.load_gather(rb0, [ridx, jnp.zeros((16,), jnp.int32)])
            dslice[pl.ds(p * B + k * 16, 16)] = dv
        return 0
    lax.fori_loop(0, NP, dext, 0)

    def gath_start(b, par, chunk):
        for j in range(B // 16):
            sl = pl.ds(j * 16, 16)
            ev = epk1d[pl.ds(b * B + j * 16, 16)]
            gbuf[sl] = (ev >> 14) + chunk * NPAD
        fill_didx(b, par)
        pltpu.async_copy(h4H.at[gbuf], rbufs[par], gsems[par])

    def gath_wait(par):
        pltpu.make_async_copy(h4H.at[gbuf], rbufs[par], gsems[par]).wait()

    def do_chunk(q, _):
        chunk = CPSC * c + q

        lax.fori_loop(0, B, zrow(rb0), 0)
        lax.fori_loop(0, NP, zacc, 0)
        plsc.subcore_barrier()

        gath_start(0, 0, chunk)

        def triple(t, _):
            for par in range(3):
                b = 3 * t + par
                nxt = (par + 1) % 3
                gath_wait(par)

                if par == 2:
                    scat_wait(nxt)
                else:
                    @pl.when(t > 0)
                    def _():
                        scat_wait(nxt)

                if par < 2:
                    gath_start(b + 1, nxt, chunk)
                else:
                    @pl.when(t < NT - 1)
                    def _():
                        gath_start(b + 1, nxt, chunk)

                @plsc.parallel_loop(0, B, 1, unroll=4)
                def scale(r, par=par, b=b):
                    wsp = wsplat(b, r)
                    for j in range(DC // 16):
                        sl = pl.ds(j * 16, 16)
                        rbufs[par][r, sl] = rbufs[par][r, sl] * wsp
                scat_start(par)
            return 0
        lax.fori_loop(0, NT, triple, 0)
        for par in (1, 2):
            scat_wait(par)
        plsc.subcore_barrier()

        def norm(p, _):
            row0 = s * RPT + p * B
            pltpu.sync_copy(acc.at[pl.ds(row0, B)], rb0)

            @plsc.parallel_loop(0, B, 1, unroll=4)
            def nrow(r, p=p):
                dsp = plsc.load_gather(dslice, [jnp.full((16,), p * B,
                                                         jnp.int32) + r])
                inv = 1.0 / (dsp + 1e-16)
                for j in range(DC // 16):
                    sl = pl.ds(j * 16, 16)
                    rb0[r, sl] = rb0[r, sl] * inv
            pltpu.sync_copy(rb0, outH.at[pl.ds(chunk * NPAD + row0, B)])
            return 0
        lax.fori_loop(0, NP, norm, 0)
        plsc.subcore_barrier()
        return 0
    lax.fori_loop(0, CPSC, do_chunk, 0)


def _sc_layer(epkH, a_s, a_d, h4):
    mesh = plsc.VectorSubcoreMesh(core_axis_name="c", subcore_axis_name="s")
    kfn = pl.kernel(
        _sc_body,
        out_type=jax.ShapeDtypeStruct((NCHUNK * NPAD, DC), jnp.float32),
        mesh=mesh,
        scratch_types=[
            pltpu.VMEM((ET,), jnp.int32),
            pltpu.VMEM((ET,), jnp.float32),
            pltpu.VMEM((NE,), jnp.float32),
            pltpu.VMEM((B, DC), jnp.float32),
            pltpu.VMEM((B, DC), jnp.float32),
            pltpu.VMEM((B, DC), jnp.float32),
            pltpu.VMEM((RPT,), jnp.float32),
            pltpu.VMEM((B,), jnp.int32),
            pltpu.VMEM((B,), jnp.int32),
            pltpu.VMEM((B,), jnp.int32),
            pltpu.VMEM((B,), jnp.int32),
            pltpu.VMEM_SHARED((NPAD, DC), jnp.float32),
            pltpu.SemaphoreType.DMA,
            pltpu.SemaphoreType.DMA,
            pltpu.SemaphoreType.DMA,
            pltpu.SemaphoreType.DMA,
            pltpu.SemaphoreType.DMA,
            pltpu.SemaphoreType.DMA,
        ],
        compiler_params=pltpu.CompilerParams(needs_layout_passes=False),
    )
    return kfn(epkH, a_s, a_d, h4)


def kernel(x, edge_index, W1, a1_src, a1_dst, b1,
           W2, a2_src, a2_dst, b2, W3, a3_src, a3_dst, b3):
    ei = edge_index.astype(jnp.int32)
    loops = jnp.arange(N, dtype=jnp.int32)
    src = jnp.concatenate([ei[0], loops])
    dst = jnp.concatenate([ei[1], loops])
    src = jnp.pad(src, (0, EPAD - src.shape[0]))
    dst = jnp.pad(dst, (0, EPAD - dst.shape[0]))
    epkH = (src * PACK + dst).reshape(16, ET)

    x_pad = jnp.pad(x, ((0, NPAD - N), (0, 0)))

    h4, a_s, a_d = _matmul(x_pad, W1, a1_src, a1_dst, b1)
    u1 = _sc_layer(epkH, a_s, a_d, h4)
    u1c = u1.reshape(NCHUNK, NPAD, DC)
    h4, a_s, a_d = _matmul(u1c, W2, a2_src, a2_dst, b1)
    u2 = _sc_layer(epkH, a_s, a_d, h4)
    u2c = u2.reshape(NCHUNK, NPAD, DC)
    h4, a_s, a_d = _matmul(u2c, W3, a3_src, a3_dst, b2)
    u3 = _sc_layer(epkH, a_s, a_d, h4)
    u3c = u3.reshape(NCHUNK, NPAD, DC)

    out = _jk_max(u1c, u2c, u3c, b1, b2, b3)
    return out[:N]


# speedup vs baseline: 9.4175x; 1.0715x over previous
import functools

import jax
import jax.numpy as jnp
from jax import lax
from jax.experimental import pallas as pl
from jax.experimental.pallas import tpu as pltpu
from jax.experimental.pallas import tpu_sc as plsc

N = 10000
NPAD = 10240
D = 512
NCHUNK = 4
DC = 128
CPSC = NCHUNK // 2
EPAD = 172032
ET = EPAD // 16
B = 64
NB = ET // B
NT = NB // 3
RPT = NPAD // 16
NP = RPT // B
NE = 1280
PACK = 16384


def _mm_body(chunked_in, x_ref, w_ref, a2_ref, b_ref, h4_ref, aa_ref):
    if chunked_in:
        x = jnp.concatenate([x_ref[c] for c in range(NCHUNK)], axis=1)
        x = jnp.maximum(x + b_ref[0], 0.0)
    else:
        x = x_ref[...]
    h = jnp.dot(x, w_ref[...],
                preferred_element_type=jnp.float32,
                precision=lax.Precision.HIGHEST)
    for c in range(NCHUNK):
        h4_ref[c] = h[:, c * DC:(c + 1) * DC]
    aa_ref[...] = jnp.dot(h, a2_ref[...],
                          preferred_element_type=jnp.float32,
                          precision=lax.Precision.HIGHEST)


def _matmul(x_in, W, a_src, a_dst, b_prev):
    chunked_in = x_in.ndim == 3
    a2 = jnp.stack([a_src, a_dst] + [jnp.zeros_like(a_src)] * 6, axis=1)
    blk = 640
    grid = NPAD // blk
    if chunked_in:
        x_spec = pl.BlockSpec((NCHUNK, blk, DC), lambda i: (0, i, 0))
        din = D
    else:
        din = x_in.shape[1]
        x_spec = pl.BlockSpec((blk, din), lambda i: (i, 0))
    h4, aa = pl.pallas_call(
        functools.partial(_mm_body, chunked_in),
        grid=(grid,),
        in_specs=[
            x_spec,
            pl.BlockSpec((din, D), lambda i: (0, 0)),
            pl.BlockSpec((D, 8), lambda i: (0, 0)),
            pl.BlockSpec((1, D), lambda i: (0, 0)),
        ],
        out_specs=[
            pl.BlockSpec((NCHUNK, blk, DC), lambda i: (0, i, 0)),
            pl.BlockSpec((blk, 8), lambda i: (i, 0)),
        ],
        out_shape=[
            jax.ShapeDtypeStruct((NCHUNK, NPAD, DC), jnp.float32),
            jax.ShapeDtypeStruct((NPAD, 8), jnp.float32),
        ],
    )(x_in, W, a2, b_prev.reshape(1, D))
    return jnp.reshape(h4, (NCHUNK * NPAD, DC)), aa[:, 0], aa[:, 1]


def _jk_body(u1, u2, u3, b1, b2, b3, o):
    outs = []
    for u, b in ((u1, b1), (u2, b2), (u3, b3)):
        xc = jnp.concatenate([u[c] for c in range(NCHUNK)], axis=1)
        outs.append(jnp.maximum(xc + b[0], 0.0))
    o[...] = jnp.maximum(jnp.maximum(outs[0], outs[1]), outs[2])


def _jk_max(u1, u2, u3, b1, b2, b3):
    blk = 640
    grid = NPAD // blk
    uspec = pl.BlockSpec((NCHUNK, blk, DC), lambda i: (0, i, 0))
    bspec = pl.BlockSpec((1, D), lambda i: (0, 0))
    return pl.pallas_call(
        _jk_body,
        grid=(grid,),
        in_specs=[uspec, uspec, uspec, bspec, bspec, bspec],
        out_specs=pl.BlockSpec((blk, D), lambda i: (i, 0)),
        out_shape=jax.ShapeDtypeStruct((NPAD, D), jnp.float32),
    )(u1, u2, u3, b1.reshape(1, D), b2.reshape(1, D), b3.reshape(1, D))


def _sc_body(epkH, asH, adH, h4H, outH,
             epk1d, w1d, abuf, rb0, rb1, rb2, dslice, gbuf, dx0, dx1, dx2,
             acc, gs0, gs1, gs2, ss0, ss1, ss2):
    c = lax.axis_index("c")
    s = lax.axis_index("s")
    zv = jnp.zeros((16,), jnp.float32)
    ebase = s * ET
    rbufs = (rb0, rb1, rb2)
    didxs = (dx0, dx1, dx2)
    gsems = (gs0, gs1, gs2)
    ssems = (ss0, ss1, ss2)

    pltpu.sync_copy(epkH.at[s], epk1d)

    for ai, arr in ((0, asH), (1, adH)):
        for piece in range(NPAD // NE):
            pltpu.sync_copy(arr.at[pl.ds(piece * NE, NE)], abuf)
            lo = piece * NE
            first = ai == 0 and piece == 0

            def gsum(i, _, ai=ai, lo=lo, first=first):
                sl = pl.ds(i * 16, 16)
                ev = epk1d[sl]
                nv = (ev >> 14) if ai == 0 else (ev & (PACK - 1))
                iv = nv - lo
                ivc = jnp.minimum(jnp.maximum(iv, 0), NE - 1)
                av = plsc.load_gather(abuf, [ivc])
                av = jnp.where((iv >= 0) & (iv < NE), av, 0.0)
                if first:
                    w1d[sl] = av
                else:
                    w1d[sl] = w1d[sl] + av
                return 0
            lax.fori_loop(0, ET // 16, gsum, 0)

    def wfin(i, _):
        sl = pl.ds(i * 16, 16)
        t = w1d[sl]
        t = jnp.maximum(t, 0.2 * t)
        w = jnp.exp(t)
        gid = ebase + i * 16 + lax.iota(jnp.int32, 16)
        w1d[sl] = jnp.where(gid < N + 160000, w, 0.0)
        return 0
    lax.fori_loop(0, ET // 16, wfin, 0)

    def zrow(rbuf):
        def f(r, _):
            for j in range(DC // 16):
                rbuf[r, pl.ds(j * 16, 16)] = zv
            return 0
        return f

    def zacc(p, _):
        pltpu.sync_copy(rb0, acc.at[pl.ds(s * RPT + p * B, B)])
        return 0

    def fill_didx(b, par):
        for j in range(B // 16):
            sl = pl.ds(j * 16, 16)
            ev = epk1d[pl.ds(b * B + j * 16, 16)]
            didxs[par][sl] = ev & (PACK - 1)

    def wsplat(b, r):
        return plsc.load_gather(w1d, [jnp.full((16,), b * B, jnp.int32) + r])

    def scat_start(par):
        pltpu.async_copy(rbufs[par], acc.at[didxs[par]], ssems[par], add=True)

    def scat_wait(par):
        pltpu.make_async_copy(rbufs[par], acc.at[didxs[par]],
                              ssems[par]).wait()

    lax.fori_loop(0, B, zrow(rb0), 0)
    lax.fori_loop(0, NP, zacc, 0)
    plsc.subcore_barrier()

    lax.fori_loop(0, B, zrow(rb0), 0)
    lax.fori_loop(0, B, zrow(rb1), 0)
    i16 = lax.iota(jnp.int32, 16)

    def dscat(i, _):
        sl = pl.ds(i * 16, 16)
        dv = epk1d[sl] & (PACK - 1)
        w = w1d[sl]
        hi = dv >> 7
        lo = dv & 127
        m0 = hi < 64
        plsc.addupdate_scatter(rb0, [jnp.minimum(hi, 63), lo], w, mask=m0)
        plsc.addupdate_scatter(rb1, [jnp.maximum(hi - 64, 0), lo], w,
                               mask=jnp.logical_not(m0))
        return 0
    lax.fori_loop(0, ET // 16, dscat, 0)

    for j in range(B // 16):
        dx0[pl.ds(j * 16, 16)] = j * 16 + i16
        dx1[pl.ds(j * 16, 16)] = 64 + j * 16 + i16
    pltpu.sync_copy(rb0, acc.at[dx0], add=True)
    pltpu.sync_copy(rb1, acc.at[dx1], add=True)
    plsc.subcore_barrier()

    pltpu.sync_copy(acc.at[pl.ds(0, 64)], rb2)
    pltpu.sync_copy(acc.at[pl.ds(64, 16)], rb1.at[pl.ds(0, 16)])

    def dfill(i, _):
        nv = s * RPT + i * 16 + i16
        hi = nv >> 7
        lo = nv & 127
        m0 = hi < 64
        a0 = plsc.load_gather(rb2, [jnp.minimum(hi, 63), lo])
        a1 = plsc.load_gather(rb1, [jnp.maximum(hi - 64, 0), lo])
        dslice[pl.ds(i * 16, 16)] = jnp.where(m0, a0, a1)
        return 0
    lax.fori_loop(0, RPT // 16, dfill, 0)
    plsc.subcore_barrier()

    def gath_start(b, par, chunk):
        for j in range(B // 16):
            sl = pl.ds(j * 16, 16)
            ev = epk1d[pl.ds(b * B + j * 16, 16)]
            gbuf[sl] = (ev >> 14) + chunk * NPAD
        fill_didx(b, par)
        pltpu.async_copy(h4H.at[gbuf], rbufs[par], gsems[par])

    def gath_wait(par):
        pltpu.make_async_copy(h4H.at[gbuf], rbufs[par], gsems[par]).wait()

    def do_chunk(q, _):
        chunk = CPSC * c + q

        lax.fori_loop(0, B, zrow(rb0), 0)
        lax.fori_loop(0, NP, zacc, 0)
        plsc.subcore_barrier()

        gath_start(0, 0, chunk)

        def triple(t, _):
            for par in range(3):
                b = 3 * t + par
                nxt = (par + 1) % 3
                gath_wait(par)

                if par == 2:
                    scat_wait(nxt)
                else:
                    @pl.when(t > 0)
                    def _():
                        scat_wait(nxt)

                if par < 2:
                    gath_start(b + 1, nxt, chunk)
                else:
                    @pl.when(t < NT - 1)
                    def _():
                        gath_start(b + 1, nxt, chunk)

                @plsc.parallel_loop(0, B, 1, unroll=4)
                def scale(r, par=par, b=b):
                    wsp = wsplat(b, r)
                    for j in range(DC // 16):
                        sl = pl.ds(j * 16, 16)
                        rbufs[par][r, sl] = rbufs[par][r, sl] * wsp
                scat_start(par)
            return 0
        lax.fori_loop(0, NT, triple, 0)
        for par in (1, 2):
            scat_wait(par)
        plsc.subcore_barrier()

        def norm(p, _):
            row0 = s * RPT + p * B
            pltpu.sync_copy(acc.at[pl.ds(row0, B)], rb0)

            @plsc.parallel_loop(0, B, 1, unroll=4)
            def nrow(r, p=p):
                dsp = plsc.load_gather(dslice, [jnp.full((16,), p * B,
                                                         jnp.int32) + r])
                inv = 1.0 / (dsp + 1e-16)
                for j in range(DC // 16):
                    sl = pl.ds(j * 16, 16)
                    rb0[r, sl] = rb0[r, sl] * inv
            pltpu.sync_copy(rb0, outH.at[pl.ds(chunk * NPAD + row0, B)])
            return 0
        lax.fori_loop(0, NP, norm, 0)
        plsc.subcore_barrier()
        return 0
    lax.fori_loop(0, CPSC, do_chunk, 0)


def _sc_layer(epkH, a_s, a_d, h4):
    mesh = plsc.VectorSubcoreMesh(core_axis_name="c", subcore_axis_name="s")
    kfn = pl.kernel(
        _sc_body,
        out_type=jax.ShapeDtypeStruct((NCHUNK * NPAD, DC), jnp.float32),
        mesh=mesh,
        scratch_types=[
            pltpu.VMEM((ET,), jnp.int32),
            pltpu.VMEM((ET,), jnp.float32),
            pltpu.VMEM((NE,), jnp.float32),
            pltpu.VMEM((B, DC), jnp.float32),
            pltpu.VMEM((B, DC), jnp.float32),
            pltpu.VMEM((B, DC), jnp.float32),
            pltpu.VMEM((RPT,), jnp.float32),
            pltpu.VMEM((B,), jnp.int32),
            pltpu.VMEM((B,), jnp.int32),
            pltpu.VMEM((B,), jnp.int32),
            pltpu.VMEM((B,), jnp.int32),
            pltpu.VMEM_SHARED((NPAD, DC), jnp.float32),
            pltpu.SemaphoreType.DMA,
            pltpu.SemaphoreType.DMA,
            pltpu.SemaphoreType.DMA,
            pltpu.SemaphoreType.DMA,
            pltpu.SemaphoreType.DMA,
            pltpu.SemaphoreType.DMA,
        ],
        compiler_params=pltpu.CompilerParams(needs_layout_passes=False),
    )
    return kfn(epkH, a_s, a_d, h4)


def kernel(x, edge_index, W1, a1_src, a1_dst, b1,
           W2, a2_src, a2_dst, b2, W3, a3_src, a3_dst, b3):
    ei = edge_index.astype(jnp.int32)
    loops = jnp.arange(N, dtype=jnp.int32)
    src = jnp.concatenate([ei[0], loops])
    dst = jnp.concatenate([ei[1], loops])
    src = jnp.pad(src, (0, EPAD - src.shape[0]))
    dst = jnp.pad(dst, (0, EPAD - dst.shape[0]))
    epkH = (src * PACK + dst).reshape(16, ET)

    x_pad = jnp.pad(x, ((0, NPAD - N), (0, 0)))

    h4, a_s, a_d = _matmul(x_pad, W1, a1_src, a1_dst, b1)
    u1 = _sc_layer(epkH, a_s, a_d, h4)
    u1c = u1.reshape(NCHUNK, NPAD, DC)
    h4, a_s, a_d = _matmul(u1c, W2, a2_src, a2_dst, b1)
    u2 = _sc_layer(epkH, a_s, a_d, h4)
    u2c = u2.reshape(NCHUNK, NPAD, DC)
    h4, a_s, a_d = _matmul(u2c, W3, a3_src, a3_dst, b2)
    u3 = _sc_layer(epkH, a_s, a_d, h4)
    u3c = u3.reshape(NCHUNK, NPAD, DC)

    out = _jk_max(u1c, u2c, u3c, b1, b2, b3)
    return out[:N]
